# baseline (speedup 1.0000x reference)
import functools

import jax
import jax.numpy as jnp
from jax import lax
from jax.experimental import pallas as pl
from jax.experimental.pallas import tpu as pltpu
from jax.experimental.pallas import tpu_sc as plsc

NV_PREV = 10242
NV = 40962
NF = 81920
CH = 64
NVP = 41984
PR = 10496
TROWS = PR // 16
NNZ_F = 3 * NF
NNZP = NNZ_F + 256
LAP_PW = NVP // 32
FACE_PW = NF // 32
NCH_F = FACE_PW // 128
NCH_L = LAP_PW // 32

_mesh = functools.partial(
    plsc.VectorSubcoreMesh, core_axis_name="c", subcore_axis_name="s"
)
_sc_params = pltpu.CompilerParams(use_tc_tiling_on_sc=False,
                                  needs_layout_passes=False)


def _sc1_body(xt, gc, gv2, ew3, ns3, lcp, lvp, gf, lap,
              idxr, gcbuf, gbuf, ebuf, nbuf, rb, fout,
              lidx, lcbuf, lvbuf, lrb, lout,
              sem0, sem1, lsem0, lsem1):
    cid = lax.axis_index("c")
    sid = lax.axis_index("s")
    wid = sid * 2 + cid
    fsems = (sem0, sem1)
    lsems = (lsem0, lsem1)
    iota16 = lax.iota(jnp.int32, 16)
    iota3 = iota16 * 3
    iota7 = iota16 * 7

    fstart = wid * FACE_PW

    def f_prime(slot, g):
        fb = fstart + g * 128
        b3 = 3 * fb
        pltpu.sync_copy(gc.at[pl.ds(b3, 384)], gcbuf.at[slot])
        pltpu.sync_copy(gv2.at[:, pl.ds(b3, 384)], gbuf.at[slot])
        pltpu.sync_copy(ew3.at[:, pl.ds(fb, 128)], ebuf.at[slot])
        pltpu.sync_copy(ns3.at[:, pl.ds(fb, 128)], nbuf.at[slot])
        for h in range(8):
            for j in range(3):
                vj = plsc.load_gather(gcbuf.at[slot], [48 * h + iota3 + j])
                idxr[slot, j, pl.ds(16 * h, 16)] = vj
        for j in range(3):
            pltpu.async_copy(xt.at[idxr.at[slot, j]], rb.at[slot, j],
                             fsems[slot])

    def f_wait(slot):
        for j in range(3):
            pltpu.make_async_copy(xt.at[idxr.at[slot, j]], rb.at[slot, j],
                                  fsems[slot]).wait()

    def f_compute(slot, g):
        fb = fstart + g * 128

        def grp(h, c2):
            base = 48 * h
            ewl = [ebuf[slot, k, pl.ds(16 * h, 16)] for k in range(3)]
            nsl = [nbuf[slot, k, pl.ds(16 * h, 16)] for k in range(3)]
            gvl = [[plsc.load_gather(gbuf.at[slot, k], [base + iota3 + j])
                    for j in range(3)] for k in range(3)]
            wv = [ewl[0] * gvl[0][j] + ewl[1] * gvl[1][j]
                  + ewl[2] * gvl[2][j] for j in range(3)]
            wv += [nsl[0] * gvl[0][j] + nsl[1] * gvl[1][j]
                   + nsl[2] * gvl[2][j] for j in range(3)]
            for jj in range(16):
                f = h * 16 + jj
                for c in range(4):
                    col = pl.ds(16 * c, 16)
                    r0 = rb[slot, 0, f, col]
                    r1 = rb[slot, 1, f, col]
                    r2 = rb[slot, 2, f, col]
                    e = wv[0][jj] * r0 + wv[1][jj] * r1 + wv[2][jj] * r2
                    n = wv[3][jj] * r0 + wv[4][jj] * r1 + wv[5][jj] * r2
                    fout[f, col] = e
                    fout[f, pl.ds(64 + 16 * c, 16)] = n
            return c2

        lax.fori_loop(0, 8, grp, 0)
        pltpu.sync_copy(fout, gf.at[pl.ds(fb, 128)])

    f_prime(0, 0)

    def f_body(gg, carry):
        for b in range(2):
            g = gg * 2 + b

            @pl.when(g + 1 < NCH_F)
            def _():
                f_prime(1 - b, g + 1)

            f_wait(b)
            f_compute(b, g)
        return carry

    lax.fori_loop(0, NCH_F // 2, f_body, 0)

    vstart = wid * LAP_PW

    def l_prime(slot, g):
        vb = vstart + g * 32
        b7 = 7 * vb
        pltpu.sync_copy(lcp.at[pl.ds(b7, 224)], lcbuf.at[slot])
        pltpu.sync_copy(lvp.at[pl.ds(b7, 224)], lvbuf.at[slot])
        for h in range(2):
            for j in range(7):
                vj = plsc.load_gather(lcbuf.at[slot], [112 * h + iota7 + j])
                lidx[slot, j, pl.ds(16 * h, 16)] = vj
        for j in range(7):
            pltpu.async_copy(xt.at[lidx.at[slot, j]], lrb.at[slot, j],
                             lsems[slot])

    def l_wait(slot):
        for j in range(7):
            pltpu.make_async_copy(xt.at[lidx.at[slot, j]], lrb.at[slot, j],
                                  lsems[slot]).wait()

    def l_compute(slot, g):
        vb = vstart + g * 32

        def grp(h, c2):
            wv = [plsc.load_gather(lvbuf.at[slot], [112 * h + iota7 + j])
                  for j in range(7)]
            for jj in range(16):
                v = h * 16 + jj
                for c in range(4):
                    col = pl.ds(16 * c, 16)
                    acc = wv[0][jj] * lrb[slot, 0, v, col]
                    for j in range(1, 7):
                        acc = acc + wv[j][jj] * lrb[slot, j, v, col]
                    lout[v, col] = acc
            return c2

        lax.fori_loop(0, 2, grp, 0)
        pltpu.sync_copy(lout, lap.at[pl.ds(vb, 32)])

    l_prime(0, 0)

    def l_body(gg, carry):
        for b in range(2):
            g = gg * 2 + b

            @pl.when(g < NCH_L)
            def _():
                @pl.when(g + 1 < NCH_L)
                def _():
                    l_prime(1 - b, g + 1)

                l_wait(b)
                l_compute(b, g)
        return carry

    lax.fori_loop(0, (NCH_L + 1) // 2, l_body, 0)


def _sc2_body(gf, frp, fcp, fvp, bnd, gv,
              bndbuf, idxF, rfr, wvb, rowbuf, acc,
              gsem0, gsem1):
    cid = lax.axis_index("c")
    sid = lax.axis_index("s")
    wid = sid * 2 + cid
    gsems = (gsem0, gsem1)
    pltpu.sync_copy(bnd, bndbuf)
    iota16 = lax.iota(jnp.int32, 16)
    zero16 = jnp.zeros((16,), jnp.float32)

    for seg in range(2):
        g_row = wid * 2 + seg
        bv = bndbuf[pl.ds(g_row, 16)]
        t0 = bv[0]
        t1 = bv[1]
        rowbase = g_row * TROWS

        def zfill(r, c2):
            for c in range(8):
                acc[r, pl.ds(16 * c, 16)] = zero16
            return c2

        lax.fori_loop(0, TROWS, zfill, 0)

        la = t0
        lb = t1
        pa = la - lax.rem(la, 8)
        n = jnp.maximum(lb - pa, 0)
        nch = (n + 127) // 128

        def prime(slot, g):
            off = pl.multiple_of(pa + g * 128, 8)
            pltpu.sync_copy(fcp.at[pl.ds(off, 128)], idxF.at[slot])
            pltpu.sync_copy(frp.at[pl.ds(off, 128)], rfr.at[slot])
            pltpu.sync_copy(fvp.at[pl.ds(off, 128)], wvb.at[slot])
            pltpu.async_copy(gf.at[idxF.at[slot]], rowbuf.at[slot],
                             gsems[slot])

        def work(slot, g):
            pltpu.make_async_copy(gf.at[idxF.at[slot]], rowbuf.at[slot],
                                  gsems[slot]).wait()
            off = pa + g * 128

            def grp(h, c3):
                tv = off + h * 16 + iota16
                valid = (tv >= la) & (tv < lb)
                frv = rfr[slot, pl.ds(h * 16, 16)]
                rix = jnp.clip(frv - rowbase, 0, TROWS - 1)
                w = jnp.where(valid, wvb[slot, pl.ds(h * 16, 16)], 0.0)
                for jj in range(16):
                    wj = w[jj]
                    rj = rix[jj]
                    r = h * 16 + jj
                    for c in range(8):
                        col = pl.ds(16 * c, 16)
                        acc[rj, col] = acc[rj, col] + rowbuf[slot, r, col] * wj
                return c3

            lax.fori_loop(0, 8, grp, 0)

        @pl.when(nch > 0)
        def _():
            prime(0, 0)

        def chunk2(gg, c2):
            for b in range(2):
                g = gg * 2 + b

                @pl.when(g < nch)
                def _():
                    @pl.when(g + 1 < nch)
                    def _():
                        prime(1 - b, g + 1)

                    work(b, g)
            return c2

        lax.fori_loop(0, (nch + 1) // 2, chunk2, 0)
        pltpu.sync_copy(acc, gv.at[pl.ds(rowbase, TROWS)])


def _mix_body(xt_ref, lap_ref, gv_ref, cf_ref, b_ref, o_ref):
    for b in range(2):
        feat = jnp.concatenate(
            [xt_ref[:, 32 * b:32 * b + 32],
             lap_ref[:, 32 * b:32 * b + 32],
             gv_ref[:, 32 * b:32 * b + 32],
             gv_ref[:, 64 + 32 * b:96 + 32 * b]], axis=1)
        acc = lax.dot_general(cf_ref[...], feat, (((1,), (1,)), ((), ())),
                              preferred_element_type=jnp.float32)
        o_ref[b] = acc + b_ref[...]


def kernel(input, Gr, Gc, Gv, Lr, Lc, Lv, Fr, Fc, Fv, NS, EW, coeffs, bias):
    f32 = jnp.float32
    xt_core = input.transpose(2, 0, 1).reshape(NV_PREV, CH)
    xt = jnp.pad(xt_core, ((0, NVP - NV_PREV), (0, 0)), constant_values=1.0)
    gc = Gc.astype(jnp.int32)
    gv2 = Gv.reshape(3, NNZ_F)
    ew3 = EW.T
    ns3 = NS.T
    lcp = jnp.pad(Lc.astype(jnp.int32), (0, 7 * NVP - 7 * NV))
    lvp = jnp.pad(Lv, (0, 7 * NVP - 7 * NV))
    frp = jnp.pad(Fr.astype(jnp.int32), (0, NNZP - NNZ_F),
                  constant_values=NV)
    fcp = jnp.pad(Fc.astype(jnp.int32), (0, NNZP - NNZ_F))
    fvp = jnp.pad(Fv, (0, NNZP - NNZ_F))
    bounds = jnp.searchsorted(
        frp[:NNZ_F],
        jnp.arange(65, dtype=jnp.int32) * TROWS).astype(jnp.int32)
    bnd = jnp.full((80,), NNZ_F, jnp.int32).at[:65].set(bounds)
    cf = coeffs.transpose(0, 2, 1).reshape(32, 128)
    bias2 = bias.reshape(32, 1)

    sc1 = pl.kernel(
        _sc1_body,
        out_type=(jax.ShapeDtypeStruct((NF, 128), f32),
                  jax.ShapeDtypeStruct((NVP, CH), f32)),
        mesh=_mesh(),
        compiler_params=_sc_params,
        scratch_types=(
            pltpu.VMEM((2, 3, 128), jnp.int32),
            pltpu.VMEM((2, 384), jnp.int32),
            pltpu.VMEM((2, 3, 384), f32),
            pltpu.VMEM((2, 3, 128), f32),
            pltpu.VMEM((2, 3, 128), f32),
            pltpu.VMEM((2, 3, 128, CH), f32),
            pltpu.VMEM((128, 128), f32),
            pltpu.VMEM((2, 7, 32), jnp.int32),
            pltpu.VMEM((2, 224), jnp.int32),
            pltpu.VMEM((2, 224), f32),
            pltpu.VMEM((2, 7, 32, CH), f32),
            pltpu.VMEM((32, CH), f32),
            pltpu.SemaphoreType.DMA,
            pltpu.SemaphoreType.DMA,
            pltpu.SemaphoreType.DMA,
            pltpu.SemaphoreType.DMA,
        ),
    )
    gf_tab, lap = sc1(xt, gc, gv2, ew3, ns3, lcp, lvp)

    sc2 = pl.kernel(
        _sc2_body,
        out_type=jax.ShapeDtypeStruct((NVP, 128), f32),
        mesh=_mesh(),
        compiler_params=_sc_params,
        scratch_types=(
            pltpu.VMEM((80,), jnp.int32),
            pltpu.VMEM((2, 128), jnp.int32),
            pltpu.VMEM((2, 128), jnp.int32),
            pltpu.VMEM((2, 128), f32),
            pltpu.VMEM((2, 128, 128), f32),
            pltpu.VMEM((TROWS, 128), f32),
            pltpu.SemaphoreType.DMA,
            pltpu.SemaphoreType.DMA,
        ),
    )
    gv = sc2(gf_tab, frp, fcp, fvp, bnd)

    out = pl.pallas_call(
        _mix_body,
        grid=(NVP // 5248,),
        in_specs=[
            pl.BlockSpec((5248, CH), lambda i: (i, 0)),
            pl.BlockSpec((5248, CH), lambda i: (i, 0)),
            pl.BlockSpec((5248, 128), lambda i: (i, 0)),
            pl.BlockSpec((32, 128), lambda i: (0, 0)),
            pl.BlockSpec((32, 1), lambda i: (0, 0)),
        ],
        out_specs=pl.BlockSpec((2, 32, 5248), lambda i: (0, 0, i)),
        out_shape=jax.ShapeDtypeStruct((2, 32, NVP), f32),
    )(xt, lap, gv, cf, bias2)
    return out[:, :, :NV]


# speedup vs baseline: 1.4569x; 1.4569x over previous
import functools

import jax
import jax.numpy as jnp
from jax import lax
from jax.experimental import pallas as pl
from jax.experimental.pallas import tpu as pltpu
from jax.experimental.pallas import tpu_sc as plsc

NV_PREV = 10242
NV = 40962
NF = 81920
CH = 64
NVP = 41984
PR = 10496
TROWS = PR // 16
NNZ_F = 3 * NF
NNZP = NNZ_F + 256
LAP_PW = NVP // 32
FACE_PW = NF // 32
NCH_F = FACE_PW // 128
NCH_L = LAP_PW // 32

_mesh = functools.partial(
    plsc.VectorSubcoreMesh, core_axis_name="c", subcore_axis_name="s"
)
_sc_params = pltpu.CompilerParams(use_tc_tiling_on_sc=False,
                                  needs_layout_passes=False)


def _sc1_body(xt, gc, gv2, ew3, ns3, lcp, lvp, gf, lap,
              idxr, gcbuf, gbuf, ebuf, nbuf, rb, fout,
              lidx, lcbuf, lvbuf, lrb, lout,
              sem0, sem1, lsem0, lsem1):
    cid = lax.axis_index("c")
    sid = lax.axis_index("s")
    wid = sid * 2 + cid
    fsems = (sem0, sem1)
    lsems = (lsem0, lsem1)
    iota16 = lax.iota(jnp.int32, 16)
    iota3 = iota16 * 3
    iota7 = iota16 * 7

    fstart = wid * FACE_PW

    def f_prime(slot, g):
        fb = fstart + g * 128
        b3 = 3 * fb
        pltpu.sync_copy(gc.at[pl.ds(b3, 384)], gcbuf.at[slot])
        pltpu.sync_copy(gv2.at[:, pl.ds(b3, 384)], gbuf.at[slot])
        pltpu.sync_copy(ew3.at[:, pl.ds(fb, 128)], ebuf.at[slot])
        pltpu.sync_copy(ns3.at[:, pl.ds(fb, 128)], nbuf.at[slot])
        for h in range(8):
            for j in range(3):
                vj = plsc.load_gather(gcbuf.at[slot], [48 * h + iota3 + j])
                idxr[slot, j, pl.ds(16 * h, 16)] = vj
        for j in range(3):
            pltpu.async_copy(xt.at[idxr.at[slot, j]], rb.at[slot, j],
                             fsems[slot])

    def f_wait(slot):
        for j in range(3):
            pltpu.make_async_copy(xt.at[idxr.at[slot, j]], rb.at[slot, j],
                                  fsems[slot]).wait()

    def f_compute(slot, g):
        fb = fstart + g * 128

        def grp(h, c2):
            base = 48 * h
            ewl = [ebuf[slot, k, pl.ds(16 * h, 16)] for k in range(3)]
            nsl = [nbuf[slot, k, pl.ds(16 * h, 16)] for k in range(3)]
            gvl = [[plsc.load_gather(gbuf.at[slot, k], [base + iota3 + j])
                    for j in range(3)] for k in range(3)]
            wv = [ewl[0] * gvl[0][j] + ewl[1] * gvl[1][j]
                  + ewl[2] * gvl[2][j] for j in range(3)]
            wv += [nsl[0] * gvl[0][j] + nsl[1] * gvl[1][j]
                   + nsl[2] * gvl[2][j] for j in range(3)]
            for jj in range(16):
                f = h * 16 + jj
                for c in range(4):
                    col = pl.ds(16 * c, 16)
                    r0 = rb[slot, 0, f, col]
                    r1 = rb[slot, 1, f, col]
                    r2 = rb[slot, 2, f, col]
                    e = wv[0][jj] * r0 + wv[1][jj] * r1 + wv[2][jj] * r2
                    n = wv[3][jj] * r0 + wv[4][jj] * r1 + wv[5][jj] * r2
                    fout[f, col] = e
                    fout[f, pl.ds(64 + 16 * c, 16)] = n
            return c2

        lax.fori_loop(0, 8, grp, 0)
        pltpu.sync_copy(fout, gf.at[pl.ds(fb, 128)])

    f_prime(0, 0)

    def f_body(gg, carry):
        for b in range(2):
            g = gg * 2 + b

            @pl.when(g + 1 < NCH_F)
            def _():
                f_prime(1 - b, g + 1)

            f_wait(b)
            f_compute(b, g)
        return carry

    lax.fori_loop(0, NCH_F // 2, f_body, 0)

    vstart = wid * LAP_PW

    def l_prime(slot, g):
        vb = vstart + g * 32
        b7 = 7 * vb
        pltpu.sync_copy(lcp.at[pl.ds(b7, 224)], lcbuf.at[slot])
        pltpu.sync_copy(lvp.at[pl.ds(b7, 224)], lvbuf.at[slot])
        for h in range(2):
            for j in range(7):
                vj = plsc.load_gather(lcbuf.at[slot], [112 * h + iota7 + j])
                lidx[slot, j, pl.ds(16 * h, 16)] = vj
        for j in range(7):
            pltpu.async_copy(xt.at[lidx.at[slot, j]], lrb.at[slot, j],
                             lsems[slot])

    def l_wait(slot):
        for j in range(7):
            pltpu.make_async_copy(xt.at[lidx.at[slot, j]], lrb.at[slot, j],
                                  lsems[slot]).wait()

    def l_compute(slot, g):
        vb = vstart + g * 32

        def grp(h, c2):
            wv = [plsc.load_gather(lvbuf.at[slot], [112 * h + iota7 + j])
                  for j in range(7)]
            for jj in range(16):
                v = h * 16 + jj
                for c in range(4):
                    col = pl.ds(16 * c, 16)
                    acc = wv[0][jj] * lrb[slot, 0, v, col]
                    for j in range(1, 7):
                        acc = acc + wv[j][jj] * lrb[slot, j, v, col]
                    lout[v, col] = acc
            return c2

        lax.fori_loop(0, 2, grp, 0)
        pltpu.sync_copy(lout, lap.at[pl.ds(vb, 32)])

    l_prime(0, 0)

    def l_body(gg, carry):
        for b in range(2):
            g = gg * 2 + b

            @pl.when(g < NCH_L)
            def _():
                @pl.when(g + 1 < NCH_L)
                def _():
                    l_prime(1 - b, g + 1)

                l_wait(b)
                l_compute(b, g)
        return carry

    lax.fori_loop(0, (NCH_L + 1) // 2, l_body, 0)


def _sc2_body(gf, frp, fcp, fvp, bnd, gv,
              bndbuf, zbuf, idxF, rfr, wvb, rowbuf, ridx, shared,
              gsem0, gsem1, zsem):
    cid = lax.axis_index("c")
    sid = lax.axis_index("s")
    gsems = (gsem0, gsem1)
    pltpu.sync_copy(bnd, bndbuf)
    bv = bndbuf[pl.ds(0, 16)]
    zero16 = jnp.zeros((16,), jnp.float32)

    def zfill(r, c2):
        for c in range(8):
            zbuf[r, pl.ds(16 * c, 16)] = zero16
        return c2

    lax.fori_loop(0, 82, zfill, 0)
    iota16 = lax.iota(jnp.int32, 16)

    for p in range(4):
        @pl.when(cid == p // 2)
        def _():
            t0 = bv[p]
            t1 = bv[p + 1]
            vbase = p * PR

            for i in range(8):
                pltpu.async_copy(
                    zbuf, shared.at[pl.ds(sid * TROWS + 82 * i, 82)], zsem)
            for i in range(8):
                pltpu.make_async_copy(
                    zbuf, shared.at[pl.ds(sid * TROWS + 82 * i, 82)],
                    zsem).wait()
            plsc.subcore_barrier()

            share = (t1 - t0 + 15) // 16
            la = t0 + sid * share
            lb = jnp.minimum(la + share, t1)
            pa = la - lax.rem(la, 8)
            n = jnp.maximum(lb - pa, 0)
            nch = (n + 127) // 128

            def prime(slot, g):
                off = pl.multiple_of(pa + g * 128, 8)
                pltpu.sync_copy(fcp.at[pl.ds(off, 128)], idxF.at[slot])
                pltpu.sync_copy(frp.at[pl.ds(off, 128)], rfr.at[slot])
                pltpu.sync_copy(fvp.at[pl.ds(off, 128)], wvb.at[slot])
                pltpu.async_copy(gf.at[idxF.at[slot]], rowbuf.at[slot],
                                 gsems[slot])

            def work(slot, g):
                pltpu.make_async_copy(gf.at[idxF.at[slot]], rowbuf.at[slot],
                                      gsems[slot]).wait()
                off = pa + g * 128

                def grp(h, c3):
                    tv = off + h * 16 + iota16
                    valid = (tv >= la) & (tv < lb)
                    frv = rfr[slot, pl.ds(h * 16, 16)]
                    rix = jnp.clip(frv - vbase, 0, PR - 1)
                    ridx[slot, pl.ds(h * 16, 16)] = rix
                    w = jnp.where(valid, wvb[slot, pl.ds(h * 16, 16)], 0.0)
                    for jj in range(16):
                        wj = w[jj]
                        r = h * 16 + jj
                        for c in range(8):
                            col = pl.ds(16 * c, 16)
                            rowbuf[slot, r, col] = rowbuf[slot, r, col] * wj
                    return c3

                lax.fori_loop(0, 8, grp, 0)
                pltpu.sync_copy(rowbuf.at[slot], shared.at[ridx.at[slot]],
                                add=True)

            @pl.when(nch > 0)
            def _():
                prime(0, 0)

            def chunk2(gg, c2):
                for b in range(2):
                    g = gg * 2 + b

                    @pl.when(g < nch)
                    def _():
                        @pl.when(g + 1 < nch)
                        def _():
                            prime(1 - b, g + 1)

                        work(b, g)
                return c2

            lax.fori_loop(0, (nch + 1) // 2, chunk2, 0)
            plsc.subcore_barrier()
            pltpu.sync_copy(shared.at[pl.ds(sid * TROWS, TROWS)],
                            gv.at[pl.ds(vbase + sid * TROWS, TROWS)])
            plsc.subcore_barrier()


def _mix_body(xt_ref, lap_ref, gv_ref, cf_ref, b_ref, o_ref):
    for b in range(2):
        feat = jnp.concatenate(
            [xt_ref[:, 32 * b:32 * b + 32],
             lap_ref[:, 32 * b:32 * b + 32],
             gv_ref[:, 32 * b:32 * b + 32],
             gv_ref[:, 64 + 32 * b:96 + 32 * b]], axis=1)
        acc = lax.dot_general(cf_ref[...], feat, (((1,), (1,)), ((), ())),
                              preferred_element_type=jnp.float32)
        o_ref[b] = acc + b_ref[...]


def kernel(input, Gr, Gc, Gv, Lr, Lc, Lv, Fr, Fc, Fv, NS, EW, coeffs, bias):
    f32 = jnp.float32
    xt_core = input.transpose(2, 0, 1).reshape(NV_PREV, CH)
    xt = jnp.pad(xt_core, ((0, NVP - NV_PREV), (0, 0)), constant_values=1.0)
    gc = Gc.astype(jnp.int32)
    gv2 = Gv.reshape(3, NNZ_F)
    ew3 = EW.T
    ns3 = NS.T
    lcp = jnp.pad(Lc.astype(jnp.int32), (0, 7 * NVP - 7 * NV))
    lvp = jnp.pad(Lv, (0, 7 * NVP - 7 * NV))
    frp = jnp.pad(Fr.astype(jnp.int32), (0, NNZP - NNZ_F),
                  constant_values=NV)
    fcp = jnp.pad(Fc.astype(jnp.int32), (0, NNZP - NNZ_F))
    fvp = jnp.pad(Fv, (0, NNZP - NNZ_F))
    bounds = jnp.searchsorted(
        frp[:NNZ_F], jnp.arange(5, dtype=jnp.int32) * PR).astype(jnp.int32)
    bnd = jnp.full((16,), NNZ_F, jnp.int32).at[:5].set(bounds)
    cf = coeffs.transpose(0, 2, 1).reshape(32, 128)
    bias2 = bias.reshape(32, 1)

    sc1 = pl.kernel(
        _sc1_body,
        out_type=(jax.ShapeDtypeStruct((NF, 128), f32),
                  jax.ShapeDtypeStruct((NVP, CH), f32)),
        mesh=_mesh(),
        compiler_params=_sc_params,
        scratch_types=(
            pltpu.VMEM((2, 3, 128), jnp.int32),
            pltpu.VMEM((2, 384), jnp.int32),
            pltpu.VMEM((2, 3, 384), f32),
            pltpu.VMEM((2, 3, 128), f32),
            pltpu.VMEM((2, 3, 128), f32),
            pltpu.VMEM((2, 3, 128, CH), f32),
            pltpu.VMEM((128, 128), f32),
            pltpu.VMEM((2, 7, 32), jnp.int32),
            pltpu.VMEM((2, 224), jnp.int32),
            pltpu.VMEM((2, 224), f32),
            pltpu.VMEM((2, 7, 32, CH), f32),
            pltpu.VMEM((32, CH), f32),
            pltpu.SemaphoreType.DMA,
            pltpu.SemaphoreType.DMA,
            pltpu.SemaphoreType.DMA,
            pltpu.SemaphoreType.DMA,
        ),
    )
    gf_tab, lap = sc1(xt, gc, gv2, ew3, ns3, lcp, lvp)

    sc2 = pl.kernel(
        _sc2_body,
        out_type=jax.ShapeDtypeStruct((NVP, 128), f32),
        mesh=_mesh(),
        compiler_params=_sc_params,
        scratch_types=(
            pltpu.VMEM((16,), jnp.int32),
            pltpu.VMEM((82, 128), f32),
            pltpu.VMEM((2, 128), jnp.int32),
            pltpu.VMEM((2, 128), jnp.int32),
            pltpu.VMEM((2, 128), f32),
            pltpu.VMEM((2, 128, 128), f32),
            pltpu.VMEM((2, 128), jnp.int32),
            pltpu.VMEM_SHARED((PR, 128), f32),
            pltpu.SemaphoreType.DMA,
            pltpu.SemaphoreType.DMA,
            pltpu.SemaphoreType.DMA,
        ),
    )
    gv = sc2(gf_tab, frp, fcp, fvp, bnd)

    out = pl.pallas_call(
        _mix_body,
        grid=(NVP // 5248,),
        in_specs=[
            pl.BlockSpec((5248, CH), lambda i: (i, 0)),
            pl.BlockSpec((5248, CH), lambda i: (i, 0)),
            pl.BlockSpec((5248, 128), lambda i: (i, 0)),
            pl.BlockSpec((32, 128), lambda i: (0, 0)),
            pl.BlockSpec((32, 1), lambda i: (0, 0)),
        ],
        out_specs=pl.BlockSpec((2, 32, 5248), lambda i: (0, 0, i)),
        out_shape=jax.ShapeDtypeStruct((2, 32, NVP), f32),
    )(xt, lap, gv, cf, bias2)
    return out[:, :, :NV]


# speedup vs baseline: 1.4768x; 1.0136x over previous
import functools

import jax
import jax.numpy as jnp
from jax import lax
from jax.experimental import pallas as pl
from jax.experimental.pallas import tpu as pltpu
from jax.experimental.pallas import tpu_sc as plsc

NV_PREV = 10242
NV = 40962
NF = 81920
CH = 64
NVP = 41984
PR = 10496
TROWS = PR // 16
NNZ_F = 3 * NF
NNZP = NNZ_F + 256
LAP_PW = NVP // 32
FACE_PW = NF // 32
NCH_F = FACE_PW // 128
NCH_L = LAP_PW // 32

_mesh = functools.partial(
    plsc.VectorSubcoreMesh, core_axis_name="c", subcore_axis_name="s"
)
_sc_params = pltpu.CompilerParams(use_tc_tiling_on_sc=False,
                                  needs_layout_passes=False)


def _sc1_body(xt, gc, gv2, ew3, ns3, lcp, lvp, gf, lap,
              idxr, gcbuf, gbuf, ebuf, nbuf, rb, fout,
              lidx, lcbuf, lvbuf, lrb, lout,
              sem0, sem1, lsem0, lsem1):
    cid = lax.axis_index("c")
    sid = lax.axis_index("s")
    wid = sid * 2 + cid
    fsems = (sem0, sem1)
    lsems = (lsem0, lsem1)
    iota16 = lax.iota(jnp.int32, 16)
    iota3 = iota16 * 3
    iota7 = iota16 * 7

    fstart = wid * FACE_PW

    def f_prime(slot, g):
        fb = fstart + g * 128
        b3 = 3 * fb
        pltpu.sync_copy(gc.at[pl.ds(b3, 384)], gcbuf.at[slot])
        pltpu.sync_copy(gv2.at[:, pl.ds(b3, 384)], gbuf.at[slot])
        pltpu.sync_copy(ew3.at[:, pl.ds(fb, 128)], ebuf.at[slot])
        pltpu.sync_copy(ns3.at[:, pl.ds(fb, 128)], nbuf.at[slot])
        for h in range(8):
            for j in range(3):
                vj = plsc.load_gather(gcbuf.at[slot], [48 * h + iota3 + j])
                idxr[slot, j, pl.ds(16 * h, 16)] = vj
        for j in range(3):
            pltpu.async_copy(xt.at[idxr.at[slot, j]], rb.at[slot, j],
                             fsems[slot])

    def f_wait(slot):
        for j in range(3):
            pltpu.make_async_copy(xt.at[idxr.at[slot, j]], rb.at[slot, j],
                                  fsems[slot]).wait()

    def f_compute(slot, g):
        fb = fstart + g * 128

        def grp(h, c2):
            base = 48 * h
            ewl = [ebuf[slot, k, pl.ds(16 * h, 16)] for k in range(3)]
            nsl = [nbuf[slot, k, pl.ds(16 * h, 16)] for k in range(3)]
            gvl = [[plsc.load_gather(gbuf.at[slot, k], [base + iota3 + j])
                    for j in range(3)] for k in range(3)]
            wv = [ewl[0] * gvl[0][j] + ewl[1] * gvl[1][j]
                  + ewl[2] * gvl[2][j] for j in range(3)]
            wv += [nsl[0] * gvl[0][j] + nsl[1] * gvl[1][j]
                   + nsl[2] * gvl[2][j] for j in range(3)]
            for jj in range(16):
                f = h * 16 + jj
                for c in range(4):
                    col = pl.ds(16 * c, 16)
                    r0 = rb[slot, 0, f, col]
                    r1 = rb[slot, 1, f, col]
                    r2 = rb[slot, 2, f, col]
                    e = wv[0][jj] * r0 + wv[1][jj] * r1 + wv[2][jj] * r2
                    n = wv[3][jj] * r0 + wv[4][jj] * r1 + wv[5][jj] * r2
                    fout[f, col] = e
                    fout[f, pl.ds(64 + 16 * c, 16)] = n
            return c2

        lax.fori_loop(0, 8, grp, 0)
        pltpu.sync_copy(fout, gf.at[pl.ds(fb, 128)])

    f_prime(0, 0)

    def f_body(gg, carry):
        for b in range(2):
            g = gg * 2 + b

            @pl.when(g + 1 < NCH_F)
            def _():
                f_prime(1 - b, g + 1)

            f_wait(b)
            f_compute(b, g)
        return carry

    lax.fori_loop(0, NCH_F // 2, f_body, 0)

    vstart = wid * LAP_PW

    def l_prime(slot, g):
        vb = vstart + g * 32
        b7 = 7 * vb
        pltpu.sync_copy(lcp.at[pl.ds(b7, 224)], lcbuf.at[slot])
        pltpu.sync_copy(lvp.at[pl.ds(b7, 224)], lvbuf.at[slot])
        for h in range(2):
            for j in range(7):
                vj = plsc.load_gather(lcbuf.at[slot], [112 * h + iota7 + j])
                lidx[slot, j, pl.ds(16 * h, 16)] = vj
        for j in range(7):
            pltpu.async_copy(xt.at[lidx.at[slot, j]], lrb.at[slot, j],
                             lsems[slot])

    def l_wait(slot):
        for j in range(7):
            pltpu.make_async_copy(xt.at[lidx.at[slot, j]], lrb.at[slot, j],
                                  lsems[slot]).wait()

    def l_compute(slot, g):
        vb = vstart + g * 32

        def grp(h, c2):
            wv = [plsc.load_gather(lvbuf.at[slot], [112 * h + iota7 + j])
                  for j in range(7)]
            for jj in range(16):
                v = h * 16 + jj
                for c in range(4):
                    col = pl.ds(16 * c, 16)
                    acc = wv[0][jj] * lrb[slot, 0, v, col]
                    for j in range(1, 7):
                        acc = acc + wv[j][jj] * lrb[slot, j, v, col]
                    lout[v, col] = acc
            return c2

        lax.fori_loop(0, 2, grp, 0)
        pltpu.sync_copy(lout, lap.at[pl.ds(vb, 32)])

    l_prime(0, 0)

    def l_body(gg, carry):
        for b in range(2):
            g = gg * 2 + b

            @pl.when(g < NCH_L)
            def _():
                @pl.when(g + 1 < NCH_L)
                def _():
                    l_prime(1 - b, g + 1)

                l_wait(b)
                l_compute(b, g)
        return carry

    lax.fori_loop(0, (NCH_L + 1) // 2, l_body, 0)


def _sc2_body(gf, frp, fcp, fvp, bnd, gv,
              bndbuf, zbuf, idxF, rfr, wvb, rowbuf, ridx, shared,
              gsem0, gsem1, zsem):
    cid = lax.axis_index("c")
    sid = lax.axis_index("s")
    gsems = (gsem0, gsem1)
    pltpu.sync_copy(bnd, bndbuf)
    bv = bndbuf[pl.ds(0, 16)]
    zero16 = jnp.zeros((16,), jnp.float32)

    def zfill(r, c2):
        for c in range(8):
            zbuf[r, pl.ds(16 * c, 16)] = zero16
        return c2

    lax.fori_loop(0, 82, zfill, 0)
    iota16 = lax.iota(jnp.int32, 16)

    for p in range(4):
        @pl.when(cid == p // 2)
        def _():
            t0 = bv[p]
            t1 = bv[p + 1]
            vbase = p * PR

            for i in range(8):
                pltpu.async_copy(
                    zbuf, shared.at[pl.ds(sid * TROWS + 82 * i, 82)], zsem)
            for i in range(8):
                pltpu.make_async_copy(
                    zbuf, shared.at[pl.ds(sid * TROWS + 82 * i, 82)],
                    zsem).wait()
            plsc.subcore_barrier()

            share = (t1 - t0 + 15) // 16
            la = t0 + sid * share
            lb = jnp.minimum(la + share, t1)
            pa = la - lax.rem(la, 8)
            n = jnp.maximum(lb - pa, 0)
            nch = (n + 127) // 128

            def prime(slot, g):
                off = pl.multiple_of(pa + g * 128, 8)
                pltpu.sync_copy(fcp.at[pl.ds(off, 128)], idxF.at[slot])
                pltpu.sync_copy(frp.at[pl.ds(off, 128)], rfr.at[slot])
                pltpu.sync_copy(fvp.at[pl.ds(off, 128)], wvb.at[slot])
                pltpu.async_copy(gf.at[idxF.at[slot]], rowbuf.at[slot],
                                 gsems[slot])

            def work(slot, g):
                pltpu.make_async_copy(gf.at[idxF.at[slot]], rowbuf.at[slot],
                                      gsems[slot]).wait()
                off = pa + g * 128

                def grp(h, c3):
                    tv = off + h * 16 + iota16
                    valid = (tv >= la) & (tv < lb)
                    frv = rfr[slot, pl.ds(h * 16, 16)]
                    rix = jnp.clip(frv - vbase, 0, PR - 1)
                    ridx[slot, pl.ds(h * 16, 16)] = rix
                    w = jnp.where(valid, wvb[slot, pl.ds(h * 16, 16)], 0.0)
                    for jj in range(16):
                        wj = w[jj]
                        r = h * 16 + jj
                        for c in range(8):
                            col = pl.ds(16 * c, 16)
                            rowbuf[slot, r, col] = rowbuf[slot, r, col] * wj
                    return c3

                lax.fori_loop(0, 8, grp, 0)
                pltpu.sync_copy(rowbuf.at[slot], shared.at[ridx.at[slot]],
                                add=True)

            @pl.when(nch > 0)
            def _():
                prime(0, 0)

            def chunk2(gg, c2):
                for b in range(2):
                    g = gg * 2 + b

                    @pl.when(g < nch)
                    def _():
                        @pl.when(g + 1 < nch)
                        def _():
                            prime(1 - b, g + 1)

                        work(b, g)
                return c2

            lax.fori_loop(0, (nch + 1) // 2, chunk2, 0)
            plsc.subcore_barrier()
            pltpu.sync_copy(shared.at[pl.ds(sid * TROWS, TROWS)],
                            gv.at[pl.ds(vbase + sid * TROWS, TROWS)])
            plsc.subcore_barrier()


def _mix_body(xt_ref, lap_ref, gv_ref, cf_ref, b_ref, o_ref):
    for b in range(2):
        feat = jnp.concatenate(
            [xt_ref[:, 32 * b:32 * b + 32],
             lap_ref[:, 32 * b:32 * b + 32],
             gv_ref[:, 32 * b:32 * b + 32],
             gv_ref[:, 64 + 32 * b:96 + 32 * b]], axis=1)
        acc = lax.dot_general(cf_ref[...], feat, (((1,), (1,)), ((), ())),
                              preferred_element_type=jnp.float32)
        o_ref[b] = acc + b_ref[...]


def kernel(input, Gr, Gc, Gv, Lr, Lc, Lv, Fr, Fc, Fv, NS, EW, coeffs, bias):
    f32 = jnp.float32
    xt_core = input.transpose(2, 0, 1).reshape(NV_PREV, CH)
    xt = jnp.pad(xt_core, ((0, NVP - NV_PREV), (0, 0)), constant_values=1.0)
    gc = Gc.astype(jnp.int32)
    gv2 = Gv.reshape(3, NNZ_F)
    ew3 = EW.T
    ns3 = NS.T
    lcp = jnp.pad(Lc.astype(jnp.int32), (0, 7 * NVP - 7 * NV))
    lvp = jnp.pad(Lv, (0, 7 * NVP - 7 * NV))
    frp = jnp.pad(Fr.astype(jnp.int32), (0, NNZP - NNZ_F),
                  constant_values=NV)
    fcp = jnp.pad(Fc.astype(jnp.int32), (0, NNZP - NNZ_F))
    fvp = jnp.pad(Fv, (0, NNZP - NNZ_F))
    bounds = jnp.searchsorted(
        frp[:NNZ_F], jnp.arange(5, dtype=jnp.int32) * PR).astype(jnp.int32)
    bnd = jnp.full((16,), NNZ_F, jnp.int32).at[:5].set(bounds)
    cf = coeffs.transpose(0, 2, 1).reshape(32, 128)
    bias2 = bias.reshape(32, 1)

    sc1 = pl.kernel(
        _sc1_body,
        out_type=(jax.ShapeDtypeStruct((NF, 128), f32),
                  jax.ShapeDtypeStruct((NVP, CH), f32)),
        mesh=_mesh(),
        compiler_params=_sc_params,
        scratch_types=(
            pltpu.VMEM((2, 3, 128), jnp.int32),
            pltpu.VMEM((2, 384), jnp.int32),
            pltpu.VMEM((2, 3, 384), f32),
            pltpu.VMEM((2, 3, 128), f32),
            pltpu.VMEM((2, 3, 128), f32),
            pltpu.VMEM((2, 3, 128, CH), f32),
            pltpu.VMEM((128, 128), f32),
            pltpu.VMEM((2, 7, 32), jnp.int32),
            pltpu.VMEM((2, 224), jnp.int32),
            pltpu.VMEM((2, 224), f32),
            pltpu.VMEM((2, 7, 32, CH), f32),
            pltpu.VMEM((32, CH), f32),
            pltpu.SemaphoreType.DMA,
            pltpu.SemaphoreType.DMA,
            pltpu.SemaphoreType.DMA,
            pltpu.SemaphoreType.DMA,
        ),
    )
    gf_tab, lap = sc1(xt, gc, gv2, ew3, ns3, lcp, lvp)

    sc2 = pl.kernel(
        _sc2_body,
        out_type=jax.ShapeDtypeStruct((NVP, 128), f32),
        mesh=_mesh(),
        compiler_params=_sc_params,
        scratch_types=(
            pltpu.VMEM((16,), jnp.int32),
            pltpu.VMEM((82, 128), f32),
            pltpu.VMEM((2, 128), jnp.int32),
            pltpu.VMEM((2, 128), jnp.int32),
            pltpu.VMEM((2, 128), f32),
            pltpu.VMEM((2, 128, 128), f32),
            pltpu.VMEM((2, 128), jnp.int32),
            pltpu.VMEM_SHARED((PR, 128), f32),
            pltpu.SemaphoreType.DMA,
            pltpu.SemaphoreType.DMA,
            pltpu.SemaphoreType.DMA,
        ),
    )
    gv = sc2(gf_tab, frp, fcp, fvp, bnd)

    out = pl.pallas_call(
        _mix_body,
        grid=(NVP // 5248,),
        in_specs=[
            pl.BlockSpec((5248, CH), lambda i: (i, 0)),
            pl.BlockSpec((5248, CH), lambda i: (i, 0)),
            pl.BlockSpec((5248, 128), lambda i: (i, 0)),
            pl.BlockSpec((32, 128), lambda i: (0, 0)),
            pl.BlockSpec((32, 1), lambda i: (0, 0)),
        ],
        out_specs=pl.BlockSpec((2, 32, 5248), lambda i: (0, 0, i)),
        out_shape=jax.ShapeDtypeStruct((2, 32, NV), f32),
    )(xt, lap, gv, cf, bias2)
    return out
